# Initial kernel scaffold; baseline (speedup 1.0000x reference)
import jax
import jax.numpy as jnp
from jax.experimental import pallas as pl


def kernel(output, target):
    raise NotImplementedError("write your pallas kernel here")


# speedup vs baseline: 1.8221x; 1.8221x over previous
import functools

import jax
import jax.numpy as jnp
from jax.experimental import pallas as pl
from jax.experimental.pallas import tpu as pltpu

_ROWS_PER_BLOCK = 8


def _acc_kernel(x_ref, t_ref, out_ref, *, scale):
    step = pl.program_id(0)

    @pl.when(step == 0)
    def _init():
        out_ref[0, 0] = jnp.float32(0.0)
        out_ref[0, 1] = jnp.float32(0.0)

    x = x_ref[...]
    t = t_ref[...]
    rb, c = x.shape
    col = jax.lax.broadcasted_iota(jnp.int32, (rb, c), 1)
    is_t = col == t
    tv = jnp.sum(jnp.where(is_t, x, jnp.float32(0.0)), axis=1, keepdims=True)
    gt = (x > tv).astype(jnp.int32)
    tie = ((x == tv) & (col < t)).astype(jnp.int32)
    rank = jnp.sum(gt + tie, axis=1)
    s = jnp.float32(scale)
    out_ref[0, 0] += jnp.sum((rank < 1).astype(jnp.float32)) * s
    out_ref[0, 1] += jnp.sum((rank < 5).astype(jnp.float32)) * s


def kernel(output, target):
    b, c = output.shape
    rb = _ROWS_PER_BLOCK
    t2d = target.astype(jnp.int32).reshape(b, 1)
    res = pl.pallas_call(
        functools.partial(_acc_kernel, scale=100.0 / b),
        grid=(b // rb,),
        in_specs=[
            pl.BlockSpec((rb, c), lambda i: (i, 0)),
            pl.BlockSpec((rb, 1), lambda i: (i, 0)),
        ],
        out_specs=pl.BlockSpec((1, 2), lambda i: (0, 0), memory_space=pltpu.SMEM),
        out_shape=jax.ShapeDtypeStruct((1, 2), jnp.float32),
    )(output, t2d)
    return (res[0, 0], res[0, 1])
